# baseline (speedup 1.0000x reference)
import functools

import jax
import jax.numpy as jnp
from jax import lax
from jax.experimental import pallas as pl
from jax.experimental.pallas import tpu as pltpu
from jax.experimental.pallas import tpu_sc as plsc

VOCAB = 1_000_000
DIM = 64
BATCH = 4096
SEQLEN = 200

_INFO = plsc.get_sparse_core_info()
NC = _INFO.num_cores
NS = _INFO.num_subcores
NW = NC * NS
BB = BATCH // NW
NBUF = 4
L = 16


def _transpose_block(rows_v, tr_v):
    iota = lax.iota(jnp.int32, L)
    for d in range(DIM):
        col = jnp.full((L,), d, jnp.int32)
        for k in range(128 // L):
            row = iota + (k * L)
            tr_v[d // 8, d % 8, pl.ds(k * L, L)] = plsc.load_gather(
                rows_v, [row, col])


def _body(seq_hbm, table_hbm, out_hbm, idx_v, *rest):
    bufs = rest[:NBUF]
    trs = rest[NBUF:2 * NBUF]
    gsems = rest[2 * NBUF:3 * NBUF]
    osems = rest[3 * NBUF:4 * NBUF]
    wid = lax.axis_index("s") * NC + lax.axis_index("c")
    pltpu.sync_copy(seq_hbm.at[:, wid], idx_v)

    def gather(t, b):
        return pltpu.make_async_copy(
            table_hbm.at[idx_v.at[t]], bufs[b], gsems[b])

    def out_copy(t, b):
        return pltpu.make_async_copy(
            trs[b], out_hbm.at[t, :, wid], osems[b])

    for b in range(NBUF):
        gather(b, b).start()

    def step(g, _):
        for b in range(NBUF):
            t = g * NBUF + b
            gather(t, b).wait()

            @pl.when(g > 0)
            def _():
                out_copy(t - NBUF, b).wait()

            _transpose_block(bufs[b], trs[b])

            @pl.when(g + 1 < SEQLEN // NBUF)
            def _():
                gather(t + NBUF, b).start()

            out_copy(t, b).start()
        return 0

    lax.fori_loop(0, SEQLEN // NBUF, step, 0)
    for b in range(NBUF):
        out_copy(SEQLEN - NBUF + b, b).wait()


@jax.jit
def _lookup(seqT, table):
    kern = pl.kernel(
        _body,
        out_type=jax.ShapeDtypeStruct((SEQLEN, 8, NW, 8, 128), jnp.float32),
        mesh=plsc.VectorSubcoreMesh(core_axis_name="c", subcore_axis_name="s"),
        scratch_types=(
            [pltpu.VMEM((SEQLEN, BB), jnp.int32)]
            + [pltpu.VMEM((BB, DIM), jnp.float32) for _ in range(NBUF)]
            + [pltpu.VMEM((8, 8, 128), jnp.float32) for _ in range(NBUF)]
            + [pltpu.SemaphoreType.DMA for _ in range(2 * NBUF)]
        ),
        compiler_params=pltpu.CompilerParams(
            use_tc_tiling_on_sc=False, needs_layout_passes=False),
    )
    return kern(seqT, table)


def kernel(seq, table):
    seqT = seq.astype(jnp.int32).T.reshape(SEQLEN, NW, BB)
    p = _lookup(seqT, table)
    return p.transpose(2, 4, 0, 1, 3).reshape(BATCH, SEQLEN, DIM)
